# Initial kernel scaffold; baseline (speedup 1.0000x reference)
import jax
import jax.numpy as jnp
from jax.experimental import pallas as pl


def kernel(observation, barcode, h, c, W_i2h, b_i2h, W_h2h, b_h2h, W_actor, b_actor, W_critic, b_critic, dnd_keys, dnd_vals, enable_embedder_layers):
    raise NotImplementedError("write your pallas kernel here")


# speedup vs baseline: 1.1387x; 1.1387x over previous
import functools

import jax
import jax.numpy as jnp
from jax import lax
from jax.experimental import pallas as pl
from jax.experimental.pallas import tpu as pltpu
from jax.experimental.pallas import tpu_sc as plsc

N_GATES = 4
NUM_ARMS = 10
BARCODE_SIZE = 40
DIM_INPUT = NUM_ARMS + BARCODE_SIZE
DIM_HIDDEN = 64
DICT_LEN = 100000

_NC = 2
_NS = 16
_NW = _NC * _NS
_D = BARCODE_SIZE
_RPW = DICT_LEN // _NW
_NCHUNK = 5
_CROWS = _RPW // _NCHUNK
_GROUPS = 40
_BUF_WORDS = _GROUPS * 16 * _D


def _sc_scan_body(q_hbm, keys_hbm, sims_hbm, idx_hbm,
                  q_v, buf_a, buf_b, stage_s, stage_i, sem_a, sem_b):
    c = lax.axis_index("c")
    s = lax.axis_index("s")
    wid = s * _NC + c
    lo = wid * _RPW

    pltpu.sync_copy(q_hbm, q_v)

    bufs = (buf_a, buf_b)
    sems = (sem_a, sem_b)
    copies = [None, None]
    copies[0] = pltpu.async_copy(
        keys_hbm.at[pl.ds(lo * _D, _CROWS * _D)],
        buf_a.at[pl.ds(0, _CROWS * _D)], sem_a)

    lane = lax.broadcasted_iota(jnp.int32, (16,), 0)
    lane_d = lane * _D
    qd = [q_v[pl.ds(d * 16, 16)] for d in range(_D)]

    best_s = jnp.full((16,), -3.0e38, jnp.float32)
    best_i = jnp.zeros((16,), jnp.int32)

    for ci in range(_NCHUNK):
        buf = bufs[ci % 2]
        copies[ci % 2].wait()
        if ci + 1 < _NCHUNK:
            nxt = (ci + 1) % 2
            copies[nxt] = pltpu.async_copy(
                keys_hbm.at[pl.ds((lo + (ci + 1) * _CROWS) * _D, _CROWS * _D)],
                bufs[nxt].at[pl.ds(0, _CROWS * _D)], sems[nxt])
        row0 = lo + ci * _CROWS

        def group(j, carry, buf=buf, row0=row0):
            bs, bi = carry
            bvec = lane_d + j * (16 * _D)
            dot = jnp.zeros((16,), jnp.float32)
            sq = jnp.zeros((16,), jnp.float32)
            for d in range(_D):
                x = plsc.load_gather(buf, [bvec + d])
                dot = dot + x * qd[d]
                sq = sq + x * x
            iv = plsc.bitcast(sq, jnp.int32)
            y = plsc.bitcast(jnp.int32(0x5F3759DF) - (iv >> 1), jnp.float32)
            for _ in range(3):
                y = y * (1.5 - 0.5 * sq * y * y)
            sim = dot * y
            rows_local = j * 16 + lane
            sim = jnp.where(rows_local < _CROWS, sim, jnp.float32(-3.0e38))
            rid = row0 + rows_local
            upd = sim > bs
            return jnp.where(upd, sim, bs), jnp.where(upd, rid, bi)

        best_s, best_i = lax.fori_loop(0, _GROUPS, group, (best_s, best_i))

    stage_s[...] = best_s
    stage_i[...] = best_i
    pltpu.sync_copy(stage_s, sims_hbm.at[wid])
    pltpu.sync_copy(stage_i, idx_hbm.at[wid])


@functools.cache
def _sc_scan():
  return pl.kernel(
    _sc_scan_body,
    out_type=(jax.ShapeDtypeStruct((_NW, 16), jnp.float32),
              jax.ShapeDtypeStruct((_NW, 16), jnp.int32)),
    mesh=plsc.VectorSubcoreMesh(core_axis_name="c", subcore_axis_name="s",
                                num_cores=_NC, num_subcores=_NS),
    scratch_types=(
        pltpu.VMEM((_D * 16,), jnp.float32),
        pltpu.VMEM((_BUF_WORDS,), jnp.float32),
        pltpu.VMEM((_BUF_WORDS,), jnp.float32),
        pltpu.VMEM((16,), jnp.float32),
        pltpu.VMEM((16,), jnp.int32),
        pltpu.SemaphoreType.DMA,
        pltpu.SemaphoreType.DMA,
    ),
    compiler_params=pltpu.CompilerParams(needs_layout_passes=False),
  )


def _tc_main_body(x_ref, h_ref, c_ref, wit_ref, bi_ref, wht_ref, bh_ref,
                  wat_ref, ba_ref, wct_ref, bc_ref, sims_ref, idx_ref,
                  keys_hbm, vals_hbm,
                  pb_out, mt_out, ct_out, ht_out, f_out, i_out, o_out, r_out,
                  pi_out, v_out,
                  keyrow_v, valrow_v, sem_k, sem_v):
    sims = sims_ref[...]
    idxs = idx_ref[...]
    m = jnp.max(sims)
    cand = jnp.where(sims == m, idxs, jnp.int32(2 ** 30))
    gidx = jnp.min(cand)
    copy_k = pltpu.make_async_copy(keys_hbm.at[pl.ds(gidx, 1)], keyrow_v, sem_k)
    copy_v = pltpu.make_async_copy(vals_hbm.at[pl.ds(gidx, 1)], valrow_v, sem_v)
    copy_k.start()
    copy_v.start()

    x = x_ref[...]
    h2 = h_ref[...]
    c2 = c_ref[...]
    preact = (jnp.dot(x, wit_ref[...], preferred_element_type=jnp.float32)
              + bi_ref[...]
              + jnp.dot(h2, wht_ref[...], preferred_element_type=jnp.float32)
              + bh_ref[...])
    gates = jax.nn.sigmoid(preact[:, :N_GATES * DIM_HIDDEN])
    f_t = gates[:, 0:DIM_HIDDEN]
    i_t = gates[:, DIM_HIDDEN:2 * DIM_HIDDEN]
    o_t = gates[:, 2 * DIM_HIDDEN:3 * DIM_HIDDEN]
    r_t = gates[:, 3 * DIM_HIDDEN:4 * DIM_HIDDEN]
    c_new = jnp.tanh(preact[:, N_GATES * DIM_HIDDEN:])
    c_t = f_t * c2 + i_t * c_new

    copy_v.wait()
    m_t = jnp.tanh(valrow_v[...])
    c_t = c_t + r_t * m_t
    h_t = o_t * jnp.tanh(c_t)

    logits = (jnp.dot(h_t, wat_ref[...], preferred_element_type=jnp.float32)
              + ba_ref[...])
    pi = jax.nn.softmax(logits, axis=-1)
    v_t = (jnp.dot(h_t, wct_ref[...], preferred_element_type=jnp.float32)
           + bc_ref[...])

    copy_k.wait()
    pb_out[...] = keyrow_v[...]
    mt_out[...] = m_t
    ct_out[...] = c_t
    ht_out[...] = h_t
    f_out[...] = f_t
    i_out[...] = i_t
    o_out[...] = o_t
    r_out[...] = r_t
    pi_out[...] = pi
    v_out[...] = v_t


def _tc_main(x, h2, c2, wit, bi, wht, bh, wat, ba, wct, bc, sims, idxs,
             dnd_keys, dnd_vals):
    n_small = 13
    in_specs = ([pl.BlockSpec(memory_space=pltpu.MemorySpace.VMEM)] * n_small
                + [pl.BlockSpec(memory_space=pl.ANY),
                   pl.BlockSpec(memory_space=pl.ANY)])
    out_shape = (
        jax.ShapeDtypeStruct((1, BARCODE_SIZE), jnp.float32),
        jax.ShapeDtypeStruct((1, DIM_HIDDEN), jnp.float32),
        jax.ShapeDtypeStruct((1, DIM_HIDDEN), jnp.float32),
        jax.ShapeDtypeStruct((1, DIM_HIDDEN), jnp.float32),
        jax.ShapeDtypeStruct((1, DIM_HIDDEN), jnp.float32),
        jax.ShapeDtypeStruct((1, DIM_HIDDEN), jnp.float32),
        jax.ShapeDtypeStruct((1, DIM_HIDDEN), jnp.float32),
        jax.ShapeDtypeStruct((1, DIM_HIDDEN), jnp.float32),
        jax.ShapeDtypeStruct((1, NUM_ARMS), jnp.float32),
        jax.ShapeDtypeStruct((1, 1), jnp.float32),
    )
    return pl.pallas_call(
        _tc_main_body,
        in_specs=in_specs,
        out_specs=tuple(pl.BlockSpec(memory_space=pltpu.MemorySpace.VMEM)
                        for _ in out_shape),
        out_shape=out_shape,
        scratch_shapes=[
            pltpu.VMEM((1, BARCODE_SIZE), jnp.float32),
            pltpu.VMEM((1, DIM_HIDDEN), jnp.float32),
            pltpu.SemaphoreType.DMA,
            pltpu.SemaphoreType.DMA,
        ],
    )(x, h2, c2, wit, bi, wht, bh, wat, ba, wct, bc, sims, idxs,
      dnd_keys, dnd_vals)


def kernel(observation, barcode, h, c, W_i2h, b_i2h, W_h2h, b_h2h,
           W_actor, b_actor, W_critic, b_critic, dnd_keys, dnd_vals,
           enable_embedder_layers):
    h2 = h.reshape(h.shape[1], -1)
    c2 = c.reshape(c.shape[1], -1)
    obs = observation.reshape(1, NUM_ARMS)
    context = barcode.reshape(1, BARCODE_SIZE)
    x_t = jnp.concatenate((obs, context), axis=1)

    q_rep = jnp.repeat(context.reshape(BARCODE_SIZE), 16)
    keys_flat = dnd_keys.reshape(DICT_LEN * BARCODE_SIZE)
    sims, idxs = _sc_scan()(q_rep, keys_flat)

    pb, m_t, c_t, h_t, f_t, i_t, o_t, r_t, pi, v_t = _tc_main(
        x_t, h2, c2,
        W_i2h.T, b_i2h.reshape(1, -1), W_h2h.T, b_h2h.reshape(1, -1),
        W_actor.T, b_actor.reshape(1, -1), W_critic.T, b_critic.reshape(1, -1),
        sims, idxs, dnd_keys, dnd_vals)

    a_t = jax.random.categorical(jax.random.key(1234),
                                 jnp.log(pi + 1e-12), axis=-1)[0]
    prob_a_t = jnp.log(pi[0, a_t] + 1e-12)
    h_out = h_t.reshape(1, 1, DIM_HIDDEN)
    c_out = c_t.reshape(1, 1, DIM_HIDDEN)
    return (a_t, pb, prob_a_t, v_t, h_out, c_out, f_t, i_t, o_t, r_t, m_t)


# speedup vs baseline: 1.3836x; 1.2151x over previous
import functools

import jax
import jax.numpy as jnp
from jax import lax
from jax.experimental import pallas as pl
from jax.experimental.pallas import tpu as pltpu
from jax.experimental.pallas import tpu_sc as plsc

N_GATES = 4
NUM_ARMS = 10
BARCODE_SIZE = 40
DIM_INPUT = NUM_ARMS + BARCODE_SIZE
DIM_HIDDEN = 64
DICT_LEN = 100000

_NC = 2
_NS = 16
_NW = _NC * _NS
_D = BARCODE_SIZE
_RPW = DICT_LEN // _NW
_NCHUNK = 5
_CROWS = _RPW // _NCHUNK
_GROUPS = 40
_BUF_WORDS = _GROUPS * 16 * _D
_GLEN = 16 * _D


def _sc_scan_body(q_hbm, keys_hbm, sims_hbm, idx_hbm,
                  q_v, buf_a, buf_b, stage_s, stage_i, sem_a, sem_b):
    c = lax.axis_index("c")
    s = lax.axis_index("s")
    wid = s * _NC + c
    lo = wid * _RPW

    pltpu.sync_copy(q_hbm, q_v)

    bufs = (buf_a, buf_b)
    sems = (sem_a, sem_b)
    copies = [None, None]
    copies[0] = pltpu.async_copy(
        keys_hbm.at[pl.ds(lo * _D, _CROWS * _D)],
        buf_a.at[pl.ds(0, _CROWS * _D)], sem_a)

    lane = lax.broadcasted_iota(jnp.int32, (16,), 0)
    lane_d = lane * _D
    lane_dk = [lane_d + k for k in range(8)]
    qd = [q_v[pl.ds(d * 16, 16)] for d in range(_D)]

    best_s = jnp.full((16,), -3.0e38, jnp.float32)
    best_i = jnp.zeros((16,), jnp.int32)

    _SLICE = _BUF_WORDS - (_GROUPS - 1) * _GLEN - (_D - 8)

    for ci in range(_NCHUNK):
        buf = bufs[ci % 2]
        copies[ci % 2].wait()
        if ci + 1 < _NCHUNK:
            nxt = (ci + 1) % 2
            copies[nxt] = pltpu.async_copy(
                keys_hbm.at[pl.ds((lo + (ci + 1) * _CROWS) * _D, _CROWS * _D)],
                bufs[nxt].at[pl.ds(0, _CROWS * _D)], sems[nxt])
        row0 = lo + ci * _CROWS

        def group(j, carry, buf=buf, row0=row0):
            bs, bi = carry
            base = j * _GLEN
            dots = [jnp.zeros((16,), jnp.float32) for _ in range(4)]
            sqs = [jnp.zeros((16,), jnp.float32) for _ in range(4)]
            for d in range(_D):
                x = plsc.load_gather(
                    buf.at[pl.ds(base + (d // 8) * 8, _SLICE)],
                    [lane_dk[d % 8]])
                k = d % 4
                dots[k] = dots[k] + x * qd[d]
                sqs[k] = sqs[k] + x * x
            dot = (dots[0] + dots[1]) + (dots[2] + dots[3])
            sq = (sqs[0] + sqs[1]) + (sqs[2] + sqs[3])
            iv = plsc.bitcast(sq, jnp.int32)
            y = plsc.bitcast(jnp.int32(0x5F3759DF) - (iv >> 1), jnp.float32)
            for _ in range(3):
                y = y * (1.5 - 0.5 * sq * y * y)
            sim = dot * y
            rows_local = j * 16 + lane
            sim = jnp.where(rows_local < _CROWS, sim, jnp.float32(-3.0e38))
            rid = row0 + rows_local
            upd = sim > bs
            return jnp.where(upd, sim, bs), jnp.where(upd, rid, bi)

        best_s, best_i = lax.fori_loop(0, _GROUPS, group, (best_s, best_i))

    stage_s[...] = best_s
    stage_i[...] = best_i
    pltpu.sync_copy(stage_s, sims_hbm.at[wid])
    pltpu.sync_copy(stage_i, idx_hbm.at[wid])


@functools.cache
def _sc_scan():
  return pl.kernel(
    _sc_scan_body,
    out_type=(jax.ShapeDtypeStruct((_NW, 16), jnp.float32),
              jax.ShapeDtypeStruct((_NW, 16), jnp.int32)),
    mesh=plsc.VectorSubcoreMesh(core_axis_name="c", subcore_axis_name="s",
                                num_cores=_NC, num_subcores=_NS),
    scratch_types=(
        pltpu.VMEM((_D * 16,), jnp.float32),
        pltpu.VMEM((_BUF_WORDS,), jnp.float32),
        pltpu.VMEM((_BUF_WORDS,), jnp.float32),
        pltpu.VMEM((16,), jnp.float32),
        pltpu.VMEM((16,), jnp.int32),
        pltpu.SemaphoreType.DMA,
        pltpu.SemaphoreType.DMA,
    ),
    compiler_params=pltpu.CompilerParams(needs_layout_passes=False,
                                         disable_bounds_checks=True),
  )


def _tc_gates_body(x_ref, h_ref, c_ref, wit_ref, bi_ref, wht_ref, bh_ref,
                   sims_ref, idx_ref,
                   gidx_out, cpart_out, f_out, i_out, o_out, r_out):
    sims = sims_ref[...]
    idxs = idx_ref[...]
    m = jnp.max(sims)
    cand = jnp.where(sims == m, idxs, jnp.int32(2 ** 30))
    gidx_out[...] = jnp.min(cand).reshape(1, 1)

    x = x_ref[...]
    h2 = h_ref[...]
    c2 = c_ref[...]
    preact = (jnp.dot(x, wit_ref[...], preferred_element_type=jnp.float32)
              + bi_ref[...]
              + jnp.dot(h2, wht_ref[...], preferred_element_type=jnp.float32)
              + bh_ref[...])
    gates = jax.nn.sigmoid(preact[:, :N_GATES * DIM_HIDDEN])
    f_t = gates[:, 0:DIM_HIDDEN]
    i_t = gates[:, DIM_HIDDEN:2 * DIM_HIDDEN]
    c_new = jnp.tanh(preact[:, N_GATES * DIM_HIDDEN:])
    cpart_out[...] = f_t * c2 + i_t * c_new
    f_out[...] = f_t
    i_out[...] = i_t
    o_out[...] = gates[:, 2 * DIM_HIDDEN:3 * DIM_HIDDEN]
    r_out[...] = gates[:, 3 * DIM_HIDDEN:4 * DIM_HIDDEN]


def _tc_gates(x, h2, c2, wit, bi, wht, bh, sims, idxs):
    out_shape = (
        jax.ShapeDtypeStruct((1, 1), jnp.int32),
        jax.ShapeDtypeStruct((1, DIM_HIDDEN), jnp.float32),
        jax.ShapeDtypeStruct((1, DIM_HIDDEN), jnp.float32),
        jax.ShapeDtypeStruct((1, DIM_HIDDEN), jnp.float32),
        jax.ShapeDtypeStruct((1, DIM_HIDDEN), jnp.float32),
        jax.ShapeDtypeStruct((1, DIM_HIDDEN), jnp.float32),
    )
    return pl.pallas_call(
        _tc_gates_body,
        out_shape=out_shape,
    )(x, h2, c2, wit, bi, wht, bh, sims, idxs)


def _tc_heads_body(cpart_ref, r_ref, o_ref, mem_ref, wat_ref, ba_ref,
                   wct_ref, bc_ref,
                   mt_out, ct_out, ht_out, pi_out, v_out):
    m_t = jnp.tanh(mem_ref[...])
    c_t = cpart_ref[...] + r_ref[...] * m_t
    h_t = o_ref[...] * jnp.tanh(c_t)
    logits = (jnp.dot(h_t, wat_ref[...], preferred_element_type=jnp.float32)
              + ba_ref[...])
    pi = jax.nn.softmax(logits, axis=-1)
    v_t = (jnp.dot(h_t, wct_ref[...], preferred_element_type=jnp.float32)
           + bc_ref[...])
    mt_out[...] = m_t
    ct_out[...] = c_t
    ht_out[...] = h_t
    pi_out[...] = pi
    v_out[...] = v_t


def _tc_heads(cpart, r_t, o_t, mem, wat, ba, wct, bc):
    out_shape = (
        jax.ShapeDtypeStruct((1, DIM_HIDDEN), jnp.float32),
        jax.ShapeDtypeStruct((1, DIM_HIDDEN), jnp.float32),
        jax.ShapeDtypeStruct((1, DIM_HIDDEN), jnp.float32),
        jax.ShapeDtypeStruct((1, NUM_ARMS), jnp.float32),
        jax.ShapeDtypeStruct((1, 1), jnp.float32),
    )
    return pl.pallas_call(
        _tc_heads_body,
        out_shape=out_shape,
    )(cpart, r_t, o_t, mem, wat, ba, wct, bc)


def kernel(observation, barcode, h, c, W_i2h, b_i2h, W_h2h, b_h2h,
           W_actor, b_actor, W_critic, b_critic, dnd_keys, dnd_vals,
           enable_embedder_layers):
    h2 = h.reshape(h.shape[1], -1)
    c2 = c.reshape(c.shape[1], -1)
    obs = observation.reshape(1, NUM_ARMS)
    context = barcode.reshape(1, BARCODE_SIZE)
    x_t = jnp.concatenate((obs, context), axis=1)

    q_rep = jnp.repeat(context.reshape(BARCODE_SIZE), 16)
    keys_flat = dnd_keys.reshape(DICT_LEN * BARCODE_SIZE)
    sims, idxs = _sc_scan()(q_rep, keys_flat)

    gidx, cpart, f_t, i_t, o_t, r_t = _tc_gates(
        x_t, h2, c2,
        W_i2h.T, b_i2h.reshape(1, -1), W_h2h.T, b_h2h.reshape(1, -1),
        sims, idxs)

    g = gidx[0, 0]
    pb = lax.dynamic_slice(dnd_keys, (g, 0), (1, BARCODE_SIZE))
    mem = lax.dynamic_slice(dnd_vals, (g, 0), (1, DIM_HIDDEN))

    m_t, c_t, h_t, pi, v_t = _tc_heads(
        cpart, r_t, o_t, mem,
        W_actor.T, b_actor.reshape(1, -1), W_critic.T, b_critic.reshape(1, -1))

    a_t = jax.random.categorical(jax.random.key(1234),
                                 jnp.log(pi + 1e-12), axis=-1)[0]
    prob_a_t = jnp.log(pi[0, a_t] + 1e-12)
    h_out = h_t.reshape(1, 1, DIM_HIDDEN)
    c_out = c_t.reshape(1, 1, DIM_HIDDEN)
    return (a_t, pb, prob_a_t, v_t, h_out, c_out, f_t, i_t, o_t, r_t, m_t)


# speedup vs baseline: 3.0168x; 2.1803x over previous
import functools

import jax
import jax.numpy as jnp
from jax import lax
from jax.experimental import pallas as pl
from jax.experimental.pallas import tpu as pltpu
from jax.experimental.pallas import tpu_sc as plsc

N_GATES = 4
NUM_ARMS = 10
BARCODE_SIZE = 40
DIM_INPUT = NUM_ARMS + BARCODE_SIZE
DIM_HIDDEN = 64
DICT_LEN = 100000

_NC = 2
_NS = 16
_NW = _NC * _NS
_D = BARCODE_SIZE
_RPW = DICT_LEN // _NW
_NCHUNK = 5
_CROWS = _RPW // _NCHUNK
_GROUPS = 40
_BUF_WORDS = _GROUPS * 16 * _D
_GLEN = 16 * _D


_CKEYS = 640
_WKEYS = _NCHUNK * _CKEYS
_WSTEP = _WKEYS
_DPAD = 100096
_LASTW = _DPAD - _WKEYS


def _sc_scan_body(q_hbm, keys_hbm, sims_hbm, idx_hbm,
                  q_v, buf_a, buf_b, stage_s, stage_i, sem_a, sem_b):
    c = lax.axis_index("c")
    s = lax.axis_index("s")
    wid = s * _NC + c
    start = pl.multiple_of(jnp.minimum(wid * _WSTEP, _LASTW), 128)

    pltpu.sync_copy(q_hbm, q_v)

    bufs = (buf_a, buf_b)
    sems = (sem_a, sem_b)
    copies = [None, None]
    copies[0] = pltpu.async_copy(
        keys_hbm.at[:, pl.ds(start, _CKEYS)], buf_a, sem_a)

    lane = lax.broadcasted_iota(jnp.int32, (16,), 0)
    qd = [q_v[pl.ds(d * 16, 16)] for d in range(_D)]

    best_s = jnp.full((16,), -3.0e38, jnp.float32)
    best_i = jnp.zeros((16,), jnp.int32)

    for ci in range(_NCHUNK):
        buf = bufs[ci % 2]
        copies[ci % 2].wait()
        if ci + 1 < _NCHUNK:
            nxt = (ci + 1) % 2
            copies[nxt] = pltpu.async_copy(
                keys_hbm.at[:, pl.ds(start + (ci + 1) * _CKEYS, _CKEYS)],
                bufs[nxt], sems[nxt])
        key0 = start + ci * _CKEYS

        def group(j, carry, buf=buf, key0=key0):
            bs, bi = carry
            j16 = j * 16
            dots = [jnp.zeros((16,), jnp.float32) for _ in range(4)]
            sqs = [jnp.zeros((16,), jnp.float32) for _ in range(4)]
            for d in range(_D):
                x = buf[d, pl.ds(j16, 16)]
                k = d % 4
                dots[k] = dots[k] + x * qd[d]
                sqs[k] = sqs[k] + x * x
            dot = (dots[0] + dots[1]) + (dots[2] + dots[3])
            sq = (sqs[0] + sqs[1]) + (sqs[2] + sqs[3])
            iv = plsc.bitcast(sq, jnp.int32)
            y = plsc.bitcast(jnp.int32(0x5F3759DF) - (iv >> 1), jnp.float32)
            for _ in range(3):
                y = y * (1.5 - 0.5 * sq * y * y)
            sim = dot * y
            rid = key0 + j16 + lane
            sim = jnp.where(rid < DICT_LEN, sim, jnp.float32(-3.0e38))
            upd = sim > bs
            return jnp.where(upd, sim, bs), jnp.where(upd, rid, bi)

        best_s, best_i = lax.fori_loop(0, _GROUPS, group, (best_s, best_i))

    stage_s[...] = best_s
    stage_i[...] = best_i
    pltpu.sync_copy(stage_s, sims_hbm.at[wid])
    pltpu.sync_copy(stage_i, idx_hbm.at[wid])


@functools.cache
def _sc_scan():
  return pl.kernel(
    _sc_scan_body,
    out_type=(jax.ShapeDtypeStruct((_NW, 16), jnp.float32),
              jax.ShapeDtypeStruct((_NW, 16), jnp.int32)),
    mesh=plsc.VectorSubcoreMesh(core_axis_name="c", subcore_axis_name="s",
                                num_cores=_NC, num_subcores=_NS),
    scratch_types=(
        pltpu.VMEM((_D * 16,), jnp.float32),
        pltpu.VMEM((_D, _CKEYS), jnp.float32),
        pltpu.VMEM((_D, _CKEYS), jnp.float32),
        pltpu.VMEM((16,), jnp.float32),
        pltpu.VMEM((16,), jnp.int32),
        pltpu.SemaphoreType.DMA,
        pltpu.SemaphoreType.DMA,
    ),
    compiler_params=pltpu.CompilerParams(needs_layout_passes=False,
                                         disable_bounds_checks=True),
  )


def _tc_gates_body(x_ref, h_ref, c_ref, wit_ref, bi_ref, wht_ref, bh_ref,
                   sims_ref, idx_ref,
                   gidx_out, cpart_out, f_out, i_out, o_out, r_out):
    sims = sims_ref[...]
    idxs = idx_ref[...]
    m = jnp.max(sims)
    cand = jnp.where(sims == m, idxs, jnp.int32(2 ** 30))
    gidx_out[...] = jnp.min(cand).reshape(1, 1)

    x = x_ref[...]
    h2 = h_ref[...]
    c2 = c_ref[...]
    preact = (jnp.dot(x, wit_ref[...], preferred_element_type=jnp.float32)
              + bi_ref[...]
              + jnp.dot(h2, wht_ref[...], preferred_element_type=jnp.float32)
              + bh_ref[...])
    gates = jax.nn.sigmoid(preact[:, :N_GATES * DIM_HIDDEN])
    f_t = gates[:, 0:DIM_HIDDEN]
    i_t = gates[:, DIM_HIDDEN:2 * DIM_HIDDEN]
    c_new = jnp.tanh(preact[:, N_GATES * DIM_HIDDEN:])
    cpart_out[...] = f_t * c2 + i_t * c_new
    f_out[...] = f_t
    i_out[...] = i_t
    o_out[...] = gates[:, 2 * DIM_HIDDEN:3 * DIM_HIDDEN]
    r_out[...] = gates[:, 3 * DIM_HIDDEN:4 * DIM_HIDDEN]


def _tc_gates(x, h2, c2, wit, bi, wht, bh, sims, idxs):
    out_shape = (
        jax.ShapeDtypeStruct((1, 1), jnp.int32),
        jax.ShapeDtypeStruct((1, DIM_HIDDEN), jnp.float32),
        jax.ShapeDtypeStruct((1, DIM_HIDDEN), jnp.float32),
        jax.ShapeDtypeStruct((1, DIM_HIDDEN), jnp.float32),
        jax.ShapeDtypeStruct((1, DIM_HIDDEN), jnp.float32),
        jax.ShapeDtypeStruct((1, DIM_HIDDEN), jnp.float32),
    )
    return pl.pallas_call(
        _tc_gates_body,
        out_shape=out_shape,
    )(x, h2, c2, wit, bi, wht, bh, sims, idxs)


def _tc_heads_body(cpart_ref, r_ref, o_ref, mem_ref, wat_ref, ba_ref,
                   wct_ref, bc_ref,
                   mt_out, ct_out, ht_out, pi_out, v_out):
    m_t = jnp.tanh(mem_ref[...])
    c_t = cpart_ref[...] + r_ref[...] * m_t
    h_t = o_ref[...] * jnp.tanh(c_t)
    logits = (jnp.dot(h_t, wat_ref[...], preferred_element_type=jnp.float32)
              + ba_ref[...])
    pi = jax.nn.softmax(logits, axis=-1)
    v_t = (jnp.dot(h_t, wct_ref[...], preferred_element_type=jnp.float32)
           + bc_ref[...])
    mt_out[...] = m_t
    ct_out[...] = c_t
    ht_out[...] = h_t
    pi_out[...] = pi
    v_out[...] = v_t


def _tc_heads(cpart, r_t, o_t, mem, wat, ba, wct, bc):
    out_shape = (
        jax.ShapeDtypeStruct((1, DIM_HIDDEN), jnp.float32),
        jax.ShapeDtypeStruct((1, DIM_HIDDEN), jnp.float32),
        jax.ShapeDtypeStruct((1, DIM_HIDDEN), jnp.float32),
        jax.ShapeDtypeStruct((1, NUM_ARMS), jnp.float32),
        jax.ShapeDtypeStruct((1, 1), jnp.float32),
    )
    return pl.pallas_call(
        _tc_heads_body,
        out_shape=out_shape,
    )(cpart, r_t, o_t, mem, wat, ba, wct, bc)


def kernel(observation, barcode, h, c, W_i2h, b_i2h, W_h2h, b_h2h,
           W_actor, b_actor, W_critic, b_critic, dnd_keys, dnd_vals,
           enable_embedder_layers):
    h2 = h.reshape(h.shape[1], -1)
    c2 = c.reshape(c.shape[1], -1)
    obs = observation.reshape(1, NUM_ARMS)
    context = barcode.reshape(1, BARCODE_SIZE)
    x_t = jnp.concatenate((obs, context), axis=1)

    q_rep = jnp.repeat(context.reshape(BARCODE_SIZE), 16)
    keys_t = dnd_keys.T
    sims, idxs = _sc_scan()(q_rep, keys_t)

    gidx, cpart, f_t, i_t, o_t, r_t = _tc_gates(
        x_t, h2, c2,
        W_i2h.T, b_i2h.reshape(1, -1), W_h2h.T, b_h2h.reshape(1, -1),
        sims, idxs)

    g = gidx[0, 0]
    pb = lax.dynamic_slice(dnd_keys, (g, 0), (1, BARCODE_SIZE))
    mem = lax.dynamic_slice(dnd_vals, (g, 0), (1, DIM_HIDDEN))

    m_t, c_t, h_t, pi, v_t = _tc_heads(
        cpart, r_t, o_t, mem,
        W_actor.T, b_actor.reshape(1, -1), W_critic.T, b_critic.reshape(1, -1))

    a_t = jax.random.categorical(jax.random.key(1234),
                                 jnp.log(pi + 1e-12), axis=-1)[0]
    prob_a_t = jnp.log(pi[0, a_t] + 1e-12)
    h_out = h_t.reshape(1, 1, DIM_HIDDEN)
    c_out = c_t.reshape(1, 1, DIM_HIDDEN)
    return (a_t, pb, prob_a_t, v_t, h_out, c_out, f_t, i_t, o_t, r_t, m_t)


# speedup vs baseline: 3.6980x; 1.2258x over previous
import functools

import jax
import jax.numpy as jnp
from jax import lax
from jax.experimental import pallas as pl
from jax.experimental.pallas import tpu as pltpu
from jax.experimental.pallas import tpu_sc as plsc

N_GATES = 4
NUM_ARMS = 10
BARCODE_SIZE = 40
DIM_INPUT = NUM_ARMS + BARCODE_SIZE
DIM_HIDDEN = 64
DICT_LEN = 100000

_NC = 2
_NS = 16
_NW = _NC * _NS
_D = BARCODE_SIZE
_RPW = DICT_LEN // _NW
_NCHUNK = 5
_CROWS = _RPW // _NCHUNK
_GROUPS = 40
_BUF_WORDS = _GROUPS * 16 * _D
_GLEN = 16 * _D


_CKEYS = 640
_WKEYS = _NCHUNK * _CKEYS
_WSTEP = _WKEYS
_DPAD = 100096
_LASTW = _DPAD - _WKEYS


def _sc_scan_body(q_hbm, keys_hbm, sims_hbm, idx_hbm,
                  q_v, buf_a, buf_b, stage_s, stage_i, sem_a, sem_b):
    c = lax.axis_index("c")
    s = lax.axis_index("s")
    wid = s * _NC + c
    start = pl.multiple_of(jnp.minimum(wid * _WSTEP, _LASTW), 128)

    pltpu.sync_copy(q_hbm, q_v)

    bufs = (buf_a, buf_b)
    sems = (sem_a, sem_b)
    copies = [None, None]
    copies[0] = pltpu.async_copy(
        keys_hbm.at[:, pl.ds(start, _CKEYS)], buf_a, sem_a)

    lane = lax.broadcasted_iota(jnp.int32, (16,), 0)

    best_s = jnp.full((16,), -3.0e38, jnp.float32)
    best_i = jnp.zeros((16,), jnp.int32)

    _GPI = 4
    _ITERS = _GROUPS // _GPI

    for ci in range(_NCHUNK):
        buf = bufs[ci % 2]
        copies[ci % 2].wait()
        if ci + 1 < _NCHUNK:
            nxt = (ci + 1) % 2
            copies[nxt] = pltpu.async_copy(
                keys_hbm.at[:, pl.ds(start + (ci + 1) * _CKEYS, _CKEYS)],
                bufs[nxt], sems[nxt])
        key0 = start + ci * _CKEYS

        def block(t, carry, buf=buf, key0=key0):
            bs, bi = carry
            base = t * (16 * _GPI)
            dots = [[jnp.zeros((16,), jnp.float32) for _ in range(2)]
                    for _ in range(_GPI)]
            sqs = [[jnp.zeros((16,), jnp.float32) for _ in range(2)]
                   for _ in range(_GPI)]
            for d in range(_D):
                qv = q_v[pl.ds(d * 16, 16)]
                k = d % 2
                for g in range(_GPI):
                    x = buf[d, pl.ds(base + g * 16, 16)]
                    dots[g][k] = dots[g][k] + x * qv
                    sqs[g][k] = sqs[g][k] + x * x
            for g in range(_GPI):
                dot = dots[g][0] + dots[g][1]
                sq = sqs[g][0] + sqs[g][1]
                iv = plsc.bitcast(sq, jnp.int32)
                y = plsc.bitcast(jnp.int32(0x5F3759DF) - (iv >> 1),
                                 jnp.float32)
                for _ in range(3):
                    y = y * (1.5 - 0.5 * sq * y * y)
                sim = dot * y
                rid = key0 + base + g * 16 + lane
                sim = jnp.where(rid < DICT_LEN, sim, jnp.float32(-3.0e38))
                upd = sim > bs
                bs = jnp.where(upd, sim, bs)
                bi = jnp.where(upd, rid, bi)
            return bs, bi

        best_s, best_i = lax.fori_loop(0, _ITERS, block, (best_s, best_i))

    stage_s[...] = best_s
    stage_i[...] = best_i
    pltpu.sync_copy(stage_s, sims_hbm.at[wid])
    pltpu.sync_copy(stage_i, idx_hbm.at[wid])


@functools.cache
def _sc_scan():
  return pl.kernel(
    _sc_scan_body,
    out_type=(jax.ShapeDtypeStruct((_NW, 16), jnp.float32),
              jax.ShapeDtypeStruct((_NW, 16), jnp.int32)),
    mesh=plsc.VectorSubcoreMesh(core_axis_name="c", subcore_axis_name="s",
                                num_cores=_NC, num_subcores=_NS),
    scratch_types=(
        pltpu.VMEM((_D * 16,), jnp.float32),
        pltpu.VMEM((_D, _CKEYS), jnp.float32),
        pltpu.VMEM((_D, _CKEYS), jnp.float32),
        pltpu.VMEM((16,), jnp.float32),
        pltpu.VMEM((16,), jnp.int32),
        pltpu.SemaphoreType.DMA,
        pltpu.SemaphoreType.DMA,
    ),
    compiler_params=pltpu.CompilerParams(needs_layout_passes=False,
                                         disable_bounds_checks=True),
  )


def _tc_gates_body(x_ref, h_ref, c_ref, wit_ref, bi_ref, wht_ref, bh_ref,
                   sims_ref, idx_ref,
                   gidx_out, cpart_out, f_out, i_out, o_out, r_out):
    sims = sims_ref[...]
    idxs = idx_ref[...]
    m = jnp.max(sims)
    cand = jnp.where(sims == m, idxs, jnp.int32(2 ** 30))
    gidx_out[...] = jnp.min(cand).reshape(1, 1)

    x = x_ref[...]
    h2 = h_ref[...]
    c2 = c_ref[...]
    preact = (jnp.dot(x, wit_ref[...], preferred_element_type=jnp.float32)
              + bi_ref[...]
              + jnp.dot(h2, wht_ref[...], preferred_element_type=jnp.float32)
              + bh_ref[...])
    gates = jax.nn.sigmoid(preact[:, :N_GATES * DIM_HIDDEN])
    f_t = gates[:, 0:DIM_HIDDEN]
    i_t = gates[:, DIM_HIDDEN:2 * DIM_HIDDEN]
    c_new = jnp.tanh(preact[:, N_GATES * DIM_HIDDEN:])
    cpart_out[...] = f_t * c2 + i_t * c_new
    f_out[...] = f_t
    i_out[...] = i_t
    o_out[...] = gates[:, 2 * DIM_HIDDEN:3 * DIM_HIDDEN]
    r_out[...] = gates[:, 3 * DIM_HIDDEN:4 * DIM_HIDDEN]


def _tc_gates(x, h2, c2, wit, bi, wht, bh, sims, idxs):
    out_shape = (
        jax.ShapeDtypeStruct((1, 1), jnp.int32),
        jax.ShapeDtypeStruct((1, DIM_HIDDEN), jnp.float32),
        jax.ShapeDtypeStruct((1, DIM_HIDDEN), jnp.float32),
        jax.ShapeDtypeStruct((1, DIM_HIDDEN), jnp.float32),
        jax.ShapeDtypeStruct((1, DIM_HIDDEN), jnp.float32),
        jax.ShapeDtypeStruct((1, DIM_HIDDEN), jnp.float32),
    )
    return pl.pallas_call(
        _tc_gates_body,
        out_shape=out_shape,
    )(x, h2, c2, wit, bi, wht, bh, sims, idxs)


def _tc_heads_body(cpart_ref, r_ref, o_ref, mem_ref, wat_ref, ba_ref,
                   wct_ref, bc_ref,
                   mt_out, ct_out, ht_out, pi_out, v_out):
    m_t = jnp.tanh(mem_ref[...])
    c_t = cpart_ref[...] + r_ref[...] * m_t
    h_t = o_ref[...] * jnp.tanh(c_t)
    logits = (jnp.dot(h_t, wat_ref[...], preferred_element_type=jnp.float32)
              + ba_ref[...])
    pi = jax.nn.softmax(logits, axis=-1)
    v_t = (jnp.dot(h_t, wct_ref[...], preferred_element_type=jnp.float32)
           + bc_ref[...])
    mt_out[...] = m_t
    ct_out[...] = c_t
    ht_out[...] = h_t
    pi_out[...] = pi
    v_out[...] = v_t


def _tc_heads(cpart, r_t, o_t, mem, wat, ba, wct, bc):
    out_shape = (
        jax.ShapeDtypeStruct((1, DIM_HIDDEN), jnp.float32),
        jax.ShapeDtypeStruct((1, DIM_HIDDEN), jnp.float32),
        jax.ShapeDtypeStruct((1, DIM_HIDDEN), jnp.float32),
        jax.ShapeDtypeStruct((1, NUM_ARMS), jnp.float32),
        jax.ShapeDtypeStruct((1, 1), jnp.float32),
    )
    return pl.pallas_call(
        _tc_heads_body,
        out_shape=out_shape,
    )(cpart, r_t, o_t, mem, wat, ba, wct, bc)


def kernel(observation, barcode, h, c, W_i2h, b_i2h, W_h2h, b_h2h,
           W_actor, b_actor, W_critic, b_critic, dnd_keys, dnd_vals,
           enable_embedder_layers):
    h2 = h.reshape(h.shape[1], -1)
    c2 = c.reshape(c.shape[1], -1)
    obs = observation.reshape(1, NUM_ARMS)
    context = barcode.reshape(1, BARCODE_SIZE)
    x_t = jnp.concatenate((obs, context), axis=1)

    q_rep = jnp.repeat(context.reshape(BARCODE_SIZE), 16)
    keys_t = dnd_keys.T
    sims, idxs = _sc_scan()(q_rep, keys_t)

    gidx, cpart, f_t, i_t, o_t, r_t = _tc_gates(
        x_t, h2, c2,
        W_i2h.T, b_i2h.reshape(1, -1), W_h2h.T, b_h2h.reshape(1, -1),
        sims, idxs)

    g = gidx[0, 0]
    pb = lax.dynamic_slice(dnd_keys, (g, 0), (1, BARCODE_SIZE))
    mem = lax.dynamic_slice(dnd_vals, (g, 0), (1, DIM_HIDDEN))

    m_t, c_t, h_t, pi, v_t = _tc_heads(
        cpart, r_t, o_t, mem,
        W_actor.T, b_actor.reshape(1, -1), W_critic.T, b_critic.reshape(1, -1))

    a_t = jax.random.categorical(jax.random.key(1234),
                                 jnp.log(pi + 1e-12), axis=-1)[0]
    prob_a_t = jnp.log(pi[0, a_t] + 1e-12)
    h_out = h_t.reshape(1, 1, DIM_HIDDEN)
    c_out = c_t.reshape(1, 1, DIM_HIDDEN)
    return (a_t, pb, prob_a_t, v_t, h_out, c_out, f_t, i_t, o_t, r_t, m_t)
